# Initial kernel scaffold; baseline (speedup 1.0000x reference)
import jax
import jax.numpy as jnp
from jax.experimental import pallas as pl


def kernel(x, table):
    raise NotImplementedError("write your pallas kernel here")


# speedup vs baseline: 1.0727x; 1.0727x over previous
import jax
import jax.numpy as jnp
from jax import lax
from jax.experimental import pallas as pl
from jax.experimental.pallas import tpu as pltpu
from jax.experimental.pallas import tpu_sc as plsc

VOCAB = 1000000
EMBED_DIM = 64
BATCH = 16384
HIST = 50
NUM_IDX = BATCH * HIST

NUM_WORKERS = 32
B_PER_W = NUM_IDX // NUM_WORKERS
LANES = 16
CHUNK = 128
SUB_GATHERS = CHUNK // LANES
N_CHUNKS = B_PER_W // CHUNK


def _sc_gather(x_flat, table):
    mesh = plsc.VectorSubcoreMesh(core_axis_name="c", subcore_axis_name="s")

    @jax.named_call
    def run(table_in, idx_in):
        @pl.kernel(
            out_type=jax.ShapeDtypeStruct((NUM_IDX, EMBED_DIM), jnp.float32),
            mesh=mesh,
            compiler_params=pltpu.CompilerParams(use_tc_tiling_on_sc=False),
            scratch_types=[
                pltpu.VMEM((CHUNK,), jnp.int32),
                pltpu.VMEM((CHUNK, EMBED_DIM), jnp.float32),
                pltpu.SemaphoreType.DMA,
            ],
        )
        def gather_kernel(table_hbm, idx_hbm, out_hbm, idx_v, rows_v, sem):
            wid = lax.axis_index("s") * 2 + lax.axis_index("c")
            base = wid * B_PER_W

            @pl.loop(0, N_CHUNKS)
            def _(c):
                off = base + c * CHUNK
                pltpu.sync_copy(idx_hbm.at[pl.ds(off, CHUNK)], idx_v)
                for j in range(SUB_GATHERS):
                    idx_vec = idx_v[pl.ds(j * LANES, LANES)]
                    pltpu.async_copy(
                        table_hbm.at[idx_vec],
                        rows_v.at[pl.ds(j * LANES, LANES)],
                        sem,
                    ).wait()
                pltpu.sync_copy(rows_v, out_hbm.at[pl.ds(off, CHUNK)])

        return gather_kernel(table_in, idx_in)

    return run(table, x_flat)


@jax.jit
def kernel(x, table):
    x_flat = x.reshape(NUM_IDX).astype(jnp.int32)
    out = _sc_gather(x_flat, table)
    return out.reshape(BATCH, HIST, EMBED_DIM)


# speedup vs baseline: 1.8708x; 1.7441x over previous
import jax
import jax.numpy as jnp
from jax import lax
from jax.experimental import pallas as pl
from jax.experimental.pallas import tpu as pltpu
from jax.experimental.pallas import tpu_sc as plsc

VOCAB = 1000000
EMBED_DIM = 64
BATCH = 16384
HIST = 50
NUM_IDX = BATCH * HIST

NUM_WORKERS = 32
B_PER_W = NUM_IDX // NUM_WORKERS
GATHER_W = 128
CHUNK = 256
SUBG = CHUNK // GATHER_W
NBUF = 4
N_CHUNKS = B_PER_W // CHUNK
N_GROUPS = N_CHUNKS // NBUF


def _sc_gather(x_flat, table):
    mesh = plsc.VectorSubcoreMesh(core_axis_name="c", subcore_axis_name="s")

    @pl.kernel(
        out_type=jax.ShapeDtypeStruct((NUM_IDX, EMBED_DIM), jnp.float32),
        mesh=mesh,
        compiler_params=pltpu.CompilerParams(use_tc_tiling_on_sc=False),
        scratch_types=(
            [
                pltpu.VMEM((B_PER_W,), jnp.int32),
                pltpu.VMEM((NBUF, CHUNK, EMBED_DIM), jnp.float32),
            ]
            + [pltpu.SemaphoreType.DMA] * (2 * NBUF)
        ),
    )
    def gather_kernel(table_hbm, idx_hbm, out_hbm, idx_all, rows_v, *sems):
        gsem = sems[:NBUF]
        wsem = sems[NBUF:]
        wid = lax.axis_index("s") * 2 + lax.axis_index("c")
        base = wid * B_PER_W
        pltpu.sync_copy(idx_hbm.at[pl.ds(base, B_PER_W)], idx_all)

        def enq_gather(c, b):
            for j in range(SUBG):
                pltpu.async_copy(
                    table_hbm.at[
                        idx_all.at[pl.ds(c * CHUNK + j * GATHER_W, GATHER_W)]
                    ],
                    rows_v.at[b, pl.ds(j * GATHER_W, GATHER_W)],
                    gsem[b],
                )

        def wait_gather(c, b):
            for j in range(SUBG):
                pltpu.make_async_copy(
                    table_hbm.at[
                        idx_all.at[pl.ds(c * CHUNK + j * GATHER_W, GATHER_W)]
                    ],
                    rows_v.at[b, pl.ds(j * GATHER_W, GATHER_W)],
                    gsem[b],
                ).wait()

        def enq_write(c, b):
            pltpu.async_copy(
                rows_v.at[b],
                out_hbm.at[pl.ds(base + c * CHUNK, CHUNK)],
                wsem[b],
            )

        def wait_write(c, b):
            pltpu.make_async_copy(
                rows_v.at[b],
                out_hbm.at[pl.ds(base + c * CHUNK, CHUNK)],
                wsem[b],
            ).wait()

        for b in range(NBUF):
            enq_gather(b, b)
        for b in range(NBUF):
            wait_gather(b, b)
            enq_write(b, b)

        @pl.loop(1, N_GROUPS)
        def _(g):
            c0 = g * NBUF
            for b in range(NBUF):
                wait_write(c0 - NBUF + b, b)
                enq_gather(c0 + b, b)
            for b in range(NBUF):
                wait_gather(c0 + b, b)
                enq_write(c0 + b, b)

        for b in range(NBUF):
            wait_write(N_CHUNKS - NBUF + b, b)

    return gather_kernel(table, x_flat)


@jax.jit
def kernel(x, table):
    x_flat = x.reshape(NUM_IDX).astype(jnp.int32)
    out = _sc_gather(x_flat, table)
    return out.reshape(BATCH, HIST, EMBED_DIM)
